# baseline (speedup 1.0000x reference)
import functools

import jax
import jax.numpy as jnp
from jax import lax
from jax.experimental import pallas as pl
from jax.experimental.pallas import tpu as pltpu
from jax.experimental.pallas import tpu_sc as plsc

B = 16384
ENT_DIM = 64
REL_DIM = 64
OUT_DIM = ENT_DIM + REL_DIM + ENT_DIM

NC = 2
NS = 16
NW = NC * NS
ROWS_PER_W = B // NW
CHUNK = 128
N_CHUNKS = ROWS_PER_W // CHUNK
L = 16


_GATHER_DNUMS = lax.GatherDimensionNumbers(
    offset_dims=(), collapsed_slice_dims=(0,), start_index_map=(0,))


def _lane_shuffle(v, idx):
    return lax.gather(v, idx[:, None], _GATHER_DNUMS, (1,),
                      mode=lax.GatherScatterMode.PROMISE_IN_BOUNDS)


def _rsqrt_newton(x):
    i = lax.bitcast_convert_type(x, jnp.int32)
    i = jnp.int32(0x5F3759DF) - lax.shift_right_arithmetic(i, jnp.int32(1))
    y = lax.bitcast_convert_type(i, jnp.float32)
    half_x = x * jnp.float32(0.5)
    for _ in range(3):
        y = y * (jnp.float32(1.5) - half_x * y * y)
    return y


def _body(h_hbm, r_hbm, t_hbm, ent_hbm, rel_hbm, out_hbm,
          idx_h, idx_r, idx_t, rows_h, rows_r, rows_t, out_v,
          sem_h, sem_r, sem_t):
    wid = lax.axis_index("s") * NC + lax.axis_index("c")
    w_base = wid * ROWS_PER_W

    def chunk_body(ci, _):
        base = w_base + ci * CHUNK
        pltpu.sync_copy(h_hbm.at[pl.ds(base, CHUNK)], idx_h)
        pltpu.sync_copy(r_hbm.at[pl.ds(base, CHUNK)], idx_r)
        pltpu.sync_copy(t_hbm.at[pl.ds(base, CHUNK)], idx_t)

        cp_h = pltpu.async_copy(ent_hbm.at[idx_h], rows_h, sem_h)
        cp_r = pltpu.async_copy(rel_hbm.at[idx_r], rows_r, sem_r)
        cp_t = pltpu.async_copy(ent_hbm.at[idx_t], rows_t, sem_t)
        cp_h.wait()
        cp_r.wait()
        cp_t.wait()

        lanes = lax.iota(jnp.int32, L)

        def row_body(i, _):
            xs = []
            acc = jnp.zeros((L,), jnp.float32)
            for src in (rows_h, rows_r, rows_t):
                for c in range(ENT_DIM // L):
                    x = src[i, pl.ds(c * L, L)]
                    xs.append(x)
                    acc = acc + x * x
            for s in (8, 4, 2, 1):
                acc = acc + _lane_shuffle(acc, lanes ^ s)
            invv = _rsqrt_newton(jnp.maximum(acc, jnp.float32(1e-24)))
            for c, x in enumerate(xs):
                out_v[i, pl.ds(c * L, L)] = x * invv
            return 0

        lax.fori_loop(0, CHUNK, row_body, 0)
        pltpu.sync_copy(out_v, out_hbm.at[pl.ds(base, CHUNK)])
        return 0

    lax.fori_loop(0, N_CHUNKS, chunk_body, 0)


def kernel(h, r, t, ent_weight, rel_weight):
    k = functools.partial(
        pl.kernel,
        out_type=jax.ShapeDtypeStruct((B, OUT_DIM), jnp.float32),
        mesh=plsc.VectorSubcoreMesh(core_axis_name="c", subcore_axis_name="s"),
        compiler_params=pltpu.CompilerParams(use_tc_tiling_on_sc=False),
        scratch_types=[
            pltpu.VMEM((CHUNK,), jnp.int32),
            pltpu.VMEM((CHUNK,), jnp.int32),
            pltpu.VMEM((CHUNK,), jnp.int32),
            pltpu.VMEM((CHUNK, ENT_DIM), jnp.float32),
            pltpu.VMEM((CHUNK, REL_DIM), jnp.float32),
            pltpu.VMEM((CHUNK, ENT_DIM), jnp.float32),
            pltpu.VMEM((CHUNK, OUT_DIM), jnp.float32),
            pltpu.SemaphoreType.DMA,
            pltpu.SemaphoreType.DMA,
            pltpu.SemaphoreType.DMA,
        ],
    )(_body)
    return k(h.astype(jnp.int32), r.astype(jnp.int32), t.astype(jnp.int32),
             ent_weight, rel_weight)


# speedup vs baseline: 1.0218x; 1.0218x over previous
import functools

import jax
import jax.numpy as jnp
from jax import lax
from jax.experimental import pallas as pl
from jax.experimental.pallas import tpu as pltpu
from jax.experimental.pallas import tpu_sc as plsc

B = 16384
ENT_DIM = 64
REL_DIM = 64
OUT_DIM = ENT_DIM + REL_DIM + ENT_DIM

NC = 2
NS = 16
NW = NC * NS
ROWS_PER_W = B // NW
CHUNK = 128
N_CHUNKS = ROWS_PER_W // CHUNK
L = 16


_GATHER_DNUMS = lax.GatherDimensionNumbers(
    offset_dims=(), collapsed_slice_dims=(0,), start_index_map=(0,))


def _lane_shuffle(v, idx):
    return lax.gather(v, idx[:, None], _GATHER_DNUMS, (1,),
                      mode=lax.GatherScatterMode.PROMISE_IN_BOUNDS)


def _rsqrt_newton(x):
    i = lax.bitcast_convert_type(x, jnp.int32)
    i = jnp.int32(0x5F3759DF) - lax.shift_right_arithmetic(i, jnp.int32(1))
    y = lax.bitcast_convert_type(i, jnp.float32)
    half_x = x * jnp.float32(0.5)
    for _ in range(2):
        y = y * (jnp.float32(1.5) - half_x * y * y)
    return y


def _body(h_hbm, r_hbm, t_hbm, ent_hbm, rel_hbm, out_hbm,
          idx_h, idx_r, idx_t, rows_h, rows_r, rows_t, out_v,
          sem_h, sem_r, sem_t):
    wid = lax.axis_index("s") * NC + lax.axis_index("c")
    w_base = wid * ROWS_PER_W

    def chunk_body(ci, _):
        base = w_base + ci * CHUNK
        pltpu.sync_copy(h_hbm.at[pl.ds(base, CHUNK)], idx_h)
        pltpu.sync_copy(r_hbm.at[pl.ds(base, CHUNK)], idx_r)
        pltpu.sync_copy(t_hbm.at[pl.ds(base, CHUNK)], idx_t)

        cp_h = pltpu.async_copy(ent_hbm.at[idx_h], rows_h, sem_h)
        cp_r = pltpu.async_copy(rel_hbm.at[idx_r], rows_r, sem_r)
        cp_t = pltpu.async_copy(ent_hbm.at[idx_t], rows_t, sem_t)
        cp_h.wait()
        cp_r.wait()
        cp_t.wait()

        lanes = lax.iota(jnp.int32, L)

        @plsc.parallel_loop(0, CHUNK, step=1, unroll=8)
        def row_body(i):
            xs = []
            acc = jnp.zeros((L,), jnp.float32)
            for src in (rows_h, rows_r, rows_t):
                for c in range(ENT_DIM // L):
                    x = src[i, pl.ds(c * L, L)]
                    xs.append(x)
                    acc = acc + x * x
            for s in (8, 4, 2, 1):
                acc = acc + _lane_shuffle(acc, lanes ^ s)
            invv = _rsqrt_newton(jnp.maximum(acc, jnp.float32(1e-24)))
            for c, x in enumerate(xs):
                out_v[i, pl.ds(c * L, L)] = x * invv
        pltpu.sync_copy(out_v, out_hbm.at[pl.ds(base, CHUNK)])
        return 0

    lax.fori_loop(0, N_CHUNKS, chunk_body, 0)


def kernel(h, r, t, ent_weight, rel_weight):
    k = functools.partial(
        pl.kernel,
        out_type=jax.ShapeDtypeStruct((B, OUT_DIM), jnp.float32),
        mesh=plsc.VectorSubcoreMesh(core_axis_name="c", subcore_axis_name="s"),
        compiler_params=pltpu.CompilerParams(use_tc_tiling_on_sc=False),
        scratch_types=[
            pltpu.VMEM((CHUNK,), jnp.int32),
            pltpu.VMEM((CHUNK,), jnp.int32),
            pltpu.VMEM((CHUNK,), jnp.int32),
            pltpu.VMEM((CHUNK, ENT_DIM), jnp.float32),
            pltpu.VMEM((CHUNK, REL_DIM), jnp.float32),
            pltpu.VMEM((CHUNK, ENT_DIM), jnp.float32),
            pltpu.VMEM((CHUNK, OUT_DIM), jnp.float32),
            pltpu.SemaphoreType.DMA,
            pltpu.SemaphoreType.DMA,
            pltpu.SemaphoreType.DMA,
        ],
    )(_body)
    return k(h.astype(jnp.int32), r.astype(jnp.int32), t.astype(jnp.int32),
             ent_weight, rel_weight)


# speedup vs baseline: 1.7068x; 1.6704x over previous
import functools

import jax
import jax.numpy as jnp
from jax import lax
from jax.experimental import pallas as pl
from jax.experimental.pallas import tpu as pltpu
from jax.experimental.pallas import tpu_sc as plsc

B = 16384
ENT_DIM = 64
REL_DIM = 64
OUT_DIM = ENT_DIM + REL_DIM + ENT_DIM

NC = 2
NS = 16
NW = NC * NS
ROWS_PER_W = B // NW
CHUNK = 128
N_CHUNKS = ROWS_PER_W // CHUNK
L = 16

_GATHER_DNUMS = lax.GatherDimensionNumbers(
    offset_dims=(), collapsed_slice_dims=(0,), start_index_map=(0,))


def _lane_shuffle(v, idx):
    return lax.gather(v, idx[:, None], _GATHER_DNUMS, (1,),
                      mode=lax.GatherScatterMode.PROMISE_IN_BOUNDS)


def _rsqrt_newton(x):
    i = lax.bitcast_convert_type(x, jnp.int32)
    i = jnp.int32(0x5F3759DF) - lax.shift_right_arithmetic(i, jnp.int32(1))
    y = lax.bitcast_convert_type(i, jnp.float32)
    half_x = x * jnp.float32(0.5)
    for _ in range(2):
        y = y * (jnp.float32(1.5) - half_x * y * y)
    return y


def _body(h_hbm, r_hbm, t_hbm, ent_hbm, rel_hbm, out_hbm,
          idx_h, idx_r, idx_t, rows_h, rows_r, rows_t, out_v, sem):
    wid = lax.axis_index("s") * NC + lax.axis_index("c")
    w_base = wid * ROWS_PER_W

    def chunk_body(ci, _):
        base = w_base + ci * CHUNK
        pltpu.sync_copy(h_hbm.at[pl.ds(base, CHUNK)], idx_h)
        pltpu.sync_copy(r_hbm.at[pl.ds(base, CHUNK)], idx_r)
        pltpu.sync_copy(t_hbm.at[pl.ds(base, CHUNK)], idx_t)

        def fire(g, _):
            gbase = g * L
            ivs_h = idx_h[pl.ds(gbase, L)]
            ivs_r = idx_r[pl.ds(gbase, L)]
            ivs_t = idx_t[pl.ds(gbase, L)]
            for k in range(L):
                j = gbase + k
                pltpu.async_copy(ent_hbm.at[pl.ds(ivs_h[k], 1)],
                                 rows_h.at[pl.ds(j, 1)], sem)
                pltpu.async_copy(rel_hbm.at[pl.ds(ivs_r[k], 1)],
                                 rows_r.at[pl.ds(j, 1)], sem)
                pltpu.async_copy(ent_hbm.at[pl.ds(ivs_t[k], 1)],
                                 rows_t.at[pl.ds(j, 1)], sem)
            return 0

        lax.fori_loop(0, CHUNK // L, fire, 0)
        pltpu.make_async_copy(ent_hbm.at[pl.ds(0, CHUNK)], rows_h, sem).wait()
        pltpu.make_async_copy(rel_hbm.at[pl.ds(0, CHUNK)], rows_r, sem).wait()
        pltpu.make_async_copy(ent_hbm.at[pl.ds(0, CHUNK)], rows_t, sem).wait()

        lanes = lax.iota(jnp.int32, L)

        @plsc.parallel_loop(0, CHUNK, step=1, unroll=8)
        def row_body(i):
            xs = []
            acc = jnp.zeros((L,), jnp.float32)
            for src in (rows_h, rows_r, rows_t):
                for c in range(ENT_DIM // L):
                    x = src[i, pl.ds(c * L, L)]
                    xs.append(x)
                    acc = acc + x * x
            for s in (8, 4, 2, 1):
                acc = acc + _lane_shuffle(acc, lanes ^ s)
            invv = _rsqrt_newton(jnp.maximum(acc, jnp.float32(1e-24)))
            for c, x in enumerate(xs):
                out_v[i, pl.ds(c * L, L)] = x * invv

        pltpu.sync_copy(out_v, out_hbm.at[pl.ds(base, CHUNK)])
        return 0

    lax.fori_loop(0, N_CHUNKS, chunk_body, 0)


def kernel(h, r, t, ent_weight, rel_weight):
    k = functools.partial(
        pl.kernel,
        out_type=jax.ShapeDtypeStruct((B, OUT_DIM), jnp.float32),
        mesh=plsc.VectorSubcoreMesh(core_axis_name="c", subcore_axis_name="s"),
        compiler_params=pltpu.CompilerParams(use_tc_tiling_on_sc=True),
        scratch_types=[
            pltpu.VMEM((CHUNK,), jnp.int32),
            pltpu.VMEM((CHUNK,), jnp.int32),
            pltpu.VMEM((CHUNK,), jnp.int32),
            pltpu.VMEM((CHUNK, ENT_DIM), jnp.float32),
            pltpu.VMEM((CHUNK, REL_DIM), jnp.float32),
            pltpu.VMEM((CHUNK, ENT_DIM), jnp.float32),
            pltpu.VMEM((CHUNK, OUT_DIM), jnp.float32),
            pltpu.SemaphoreType.DMA,
        ],
    )(_body)
    return k(h.astype(jnp.int32), r.astype(jnp.int32), t.astype(jnp.int32),
             ent_weight, rel_weight)
